# baseline (speedup 1.0000x reference)
import functools

import jax
import jax.numpy as jnp
from jax import lax
from jax.experimental import pallas as pl
from jax.experimental.pallas import tpu as pltpu
from jax.experimental.pallas import tpu_sc as plsc

N = 10000
E = 320000
D = 128
NC = 2
NS = 16
NW = NC * NS
EPW = E // NW
C = 80
NCH = EPW // C
NP = 10240
RPT = NP // NS
R = 1000
GRID = N // R


def _sc_degree(dst3, z128, ones128):
  mesh = plsc.VectorSubcoreMesh(core_axis_name="c", subcore_axis_name="s")
  scratch = [
      pltpu.VMEM((NCH, C), jnp.int32),
      pltpu.VMEM((C,), jnp.int32),
      pltpu.VMEM((C, D), jnp.float32),
      pltpu.VMEM_SHARED((NP, D), jnp.float32),
  ]

  @functools.partial(
      pl.kernel, mesh=mesh,
      out_type=jax.ShapeDtypeStruct((NC, NP, D), jnp.float32),
      scratch_types=scratch)
  def k(dst_hbm, z128_hbm, ones_hbm, deg_hbm, dst_v, dst_cv, ones_v, dacc):
    c = lax.axis_index("c")
    s = lax.axis_index("s")
    wid = c * NS + s
    pltpu.sync_copy(dst_hbm.at[wid], dst_v)
    pltpu.sync_copy(ones_hbm, ones_v)
    pltpu.sync_copy(z128_hbm, dacc.at[pl.ds(s * RPT, RPT)])
    plsc.subcore_barrier()

    @pl.loop(0, NCH)
    def _(j):
      @pl.loop(0, C, step=16)
      def _(q):
        dst_cv[pl.ds(q, 16)] = dst_v[j, pl.ds(q, 16)]
      pltpu.sync_copy(ones_v, dacc.at[dst_cv], add=True)

    plsc.subcore_barrier()
    pltpu.sync_copy(dacc.at[pl.ds(s * RPT, RPT)],
                    deg_hbm.at[c].at[pl.ds(s * RPT, RPT)])

  return k(dst3, z128, ones128)


def _sc_aggregate(h, src3, dst3, z128):
  mesh = plsc.VectorSubcoreMesh(core_axis_name="c", subcore_axis_name="s")
  HC = C // 2
  scratch = [
      pltpu.VMEM((NCH, C), jnp.int32),
      pltpu.VMEM((NCH, C), jnp.int32),
      pltpu.VMEM((HC,), jnp.int32),
      pltpu.VMEM((HC,), jnp.int32),
      pltpu.VMEM((C, D), jnp.float32),
      pltpu.VMEM_SHARED((NP, D), jnp.float32),
      pltpu.SemaphoreType.DMA,
  ]

  @functools.partial(
      pl.kernel, mesh=mesh,
      out_type=jax.ShapeDtypeStruct((NC, NP, D), jnp.float32),
      scratch_types=scratch)
  def k(h_hbm, src_hbm, dst_hbm, z128_hbm, agg_hbm,
        src_v, dst_v, cv0, cv1, rows_v, acc, sem):
    rows_a = rows_v.at[pl.ds(0, HC)]
    rows_b = rows_v.at[pl.ds(HC, HC)]
    c = lax.axis_index("c")
    s = lax.axis_index("s")
    wid = c * NS + s
    pltpu.sync_copy(src_hbm.at[wid], src_v)
    pltpu.sync_copy(dst_hbm.at[wid], dst_v)
    pltpu.sync_copy(z128_hbm, acc.at[pl.ds(s * RPT, RPT)])
    plsc.subcore_barrier()

    def copy_idx(j, q0, cv):
      @pl.loop(0, HC, step=16)
      def _(q):
        cv[pl.ds(q, 16)] = dst_v[j, pl.ds(q0 + q, 16)]

    def fire(j, q0, half):
      return pltpu.async_copy(
          h_hbm.at[src_v.at[j].at[pl.ds(q0, HC)]], half, sem)

    def drain(j, q0, half):
      pltpu.make_async_copy(
          h_hbm.at[src_v.at[j].at[pl.ds(q0, HC)]], half, sem).wait()

    copy_idx(0, 0, cv0)
    fire(0, 0, rows_a)

    @pl.loop(0, NCH)
    def _(j):
      drain(j, 0, rows_a)
      copy_idx(j, HC, cv1)
      fire(j, HC, rows_b)
      pltpu.sync_copy(rows_a, acc.at[cv0], add=True)
      drain(j, HC, rows_b)
      jn = jnp.minimum(j + 1, NCH - 1)
      copy_idx(jn, 0, cv0)
      fire(jn, 0, rows_a)
      pltpu.sync_copy(rows_b, acc.at[cv1], add=True)

    drain(NCH - 1, 0, rows_a)
    plsc.subcore_barrier()
    pltpu.sync_copy(acc.at[pl.ds(s * RPT, RPT)],
                    agg_hbm.at[c].at[pl.ds(s * RPT, RPT)])

  return k(h, src3, dst3, z128)


def _tc_linear_stats(agg2, deg2, h, wlt, bl, wrt):
  def body(agg_ref, deg_ref, h_ref, wl_ref, bl_ref, wr_ref,
           z_ref, st_ref, acc_ref):
    i = pl.program_id(0)
    a = agg_ref[0] + agg_ref[1]
    dg = jnp.maximum(deg_ref[0, :, 0:1] + deg_ref[1, :, 0:1], 1.0)
    mean = a / dg
    z = (jnp.dot(mean, wl_ref[...], preferred_element_type=jnp.float32)
         + bl_ref[...]
         + jnp.dot(h_ref[...], wr_ref[...], preferred_element_type=jnp.float32))
    z_ref[...] = z

    @pl.when(i == 0)
    def _():
      acc_ref[...] = jnp.zeros_like(acc_ref)

    acc_ref[0:1, :] += jnp.sum(z, axis=0, keepdims=True)
    acc_ref[1:2, :] += jnp.sum(z * z, axis=0, keepdims=True)
    st_ref[...] = acc_ref[...]

  return pl.pallas_call(
      body,
      grid=(GRID,),
      in_specs=[
          pl.BlockSpec((NC, R, D), lambda i: (0, i, 0)),
          pl.BlockSpec((NC, R, D), lambda i: (0, i, 0)),
          pl.BlockSpec((R, D), lambda i: (i, 0)),
          pl.BlockSpec((D, D), lambda i: (0, 0)),
          pl.BlockSpec((1, D), lambda i: (0, 0)),
          pl.BlockSpec((D, D), lambda i: (0, 0)),
      ],
      out_specs=[
          pl.BlockSpec((R, D), lambda i: (i, 0)),
          pl.BlockSpec((2, D), lambda i: (0, 0)),
      ],
      out_shape=[
          jax.ShapeDtypeStruct((N, D), jnp.float32),
          jax.ShapeDtypeStruct((2, D), jnp.float32),
      ],
      scratch_shapes=[pltpu.VMEM((2, D), jnp.float32)],
  )(agg2, deg2, h, wlt, bl, wrt)


def _tc_bn_relu(z, st, g, be):
  def body(z_ref, st_ref, g_ref, be_ref, o_ref):
    mu = st_ref[0:1, :] * (1.0 / N)
    var = st_ref[1:2, :] * (1.0 / N) - mu * mu
    inv = lax.rsqrt(var + 1e-5)
    o_ref[...] = jnp.maximum(
        g_ref[...] * (z_ref[...] - mu) * inv + be_ref[...], 0.0)

  return pl.pallas_call(
      body,
      grid=(GRID,),
      in_specs=[
          pl.BlockSpec((R, D), lambda i: (i, 0)),
          pl.BlockSpec((2, D), lambda i: (0, 0)),
          pl.BlockSpec((1, D), lambda i: (0, 0)),
          pl.BlockSpec((1, D), lambda i: (0, 0)),
      ],
      out_specs=pl.BlockSpec((R, D), lambda i: (i, 0)),
      out_shape=jax.ShapeDtypeStruct((N, D), jnp.float32),
  )(z, st, g, be)


def _tc_classifier(h, w1t, b1, w2t, b2):
  def body(h_ref, w1_ref, b1_ref, w2_ref, b2_ref, p_ref):
    t = jnp.maximum(
        jnp.dot(h_ref[...], w1_ref[...], preferred_element_type=jnp.float32)
        + b1_ref[...], 0.0)
    lg = (jnp.dot(t, w2_ref[...], preferred_element_type=jnp.float32)
          + b2_ref[...])
    m = jnp.max(lg, axis=1, keepdims=True)
    e = jnp.exp(lg - m)
    p_ref[...] = e / jnp.sum(e, axis=1, keepdims=True)

  return pl.pallas_call(
      body,
      grid=(GRID,),
      in_specs=[
          pl.BlockSpec((R, D), lambda i: (i, 0)),
          pl.BlockSpec((D, 64), lambda i: (0, 0)),
          pl.BlockSpec((1, 64), lambda i: (0, 0)),
          pl.BlockSpec((64, 10), lambda i: (0, 0)),
          pl.BlockSpec((1, 10), lambda i: (0, 0)),
      ],
      out_specs=pl.BlockSpec((R, 10), lambda i: (i, 0)),
      out_shape=jax.ShapeDtypeStruct((N, 10), jnp.float32),
  )(h, w1t, b1, w2t, b2)


def kernel(x, edge_index, Wl0, bl0, Wr0, g0, be0, Wl1, bl1, Wr1, g1, be1,
           Wl2, bl2, Wr2, g2, be2, W1, b1, W2, b2):
  src3 = edge_index[0].reshape(NW, NCH, C)
  dst3 = edge_index[1].reshape(NW, NCH, C)
  z128 = jnp.zeros((RPT, D), jnp.float32)
  ones128 = jnp.ones((C, D), jnp.float32)

  deg2 = _sc_degree(dst3, z128, ones128)
  h = x
  for (Wl, bl, Wr, g, be) in ((Wl0, bl0, Wr0, g0, be0),
                              (Wl1, bl1, Wr1, g1, be1),
                              (Wl2, bl2, Wr2, g2, be2)):
    agg2 = _sc_aggregate(h, src3, dst3, z128)
    z, st = _tc_linear_stats(agg2, deg2, h, Wl.T, bl.reshape(1, D),
                             Wr.T)
    h = _tc_bn_relu(z, st, g.reshape(1, D), be.reshape(1, D))

  p = _tc_classifier(h, W1.T, b1.reshape(1, 64), W2.T, b2.reshape(1, 10))
  return (h, p)


# speedup vs baseline: 1.1974x; 1.1974x over previous
import functools

import jax
import jax.numpy as jnp
from jax import lax
from jax.experimental import pallas as pl
from jax.experimental.pallas import tpu as pltpu
from jax.experimental.pallas import tpu_sc as plsc

N = 10000
E = 320000
D = 128
NC = 2
NS = 16
NW = NC * NS
EPW = E // NW
C = 80
NCH = EPW // C
CB = 112
NCHB = EPW // CB
CT = EPW - NCHB * CB
NP = 10240
RPT = NP // NS
R = 1000
GRID = N // R


def _sc_degree(dst3, z128, ones128):
  mesh = plsc.VectorSubcoreMesh(core_axis_name="c", subcore_axis_name="s")
  scratch = [
      pltpu.VMEM((NCH, C), jnp.int32),
      pltpu.VMEM((C,), jnp.int32),
      pltpu.VMEM((C, D), jnp.float32),
      pltpu.VMEM_SHARED((NP, D), jnp.float32),
  ]

  @functools.partial(
      pl.kernel, mesh=mesh,
      out_type=jax.ShapeDtypeStruct((NC, NP, D), jnp.float32),
      scratch_types=scratch)
  def k(dst_hbm, z128_hbm, ones_hbm, deg_hbm, dst_v, dst_cv, ones_v, dacc):
    c = lax.axis_index("c")
    s = lax.axis_index("s")
    wid = c * NS + s
    pltpu.sync_copy(dst_hbm.at[wid], dst_v)
    pltpu.sync_copy(ones_hbm, ones_v)
    pltpu.sync_copy(z128_hbm, dacc.at[pl.ds(s * RPT, RPT)])
    plsc.subcore_barrier()

    @pl.loop(0, NCH)
    def _(j):
      @pl.loop(0, C, step=16)
      def _(q):
        dst_cv[pl.ds(q, 16)] = dst_v[j, pl.ds(q, 16)]
      pltpu.sync_copy(ones_v, dacc.at[dst_cv], add=True)

    plsc.subcore_barrier()
    pltpu.sync_copy(dacc.at[pl.ds(s * RPT, RPT)],
                    deg_hbm.at[c].at[pl.ds(s * RPT, RPT)])

  return k(dst3, z128, ones128)


def _sc_aggregate(h, src3, dst3, z128):
  mesh = plsc.VectorSubcoreMesh(core_axis_name="c", subcore_axis_name="s")
  scratch = [
      pltpu.VMEM((EPW,), jnp.int32),
      pltpu.VMEM((EPW,), jnp.int32),
      pltpu.VMEM((CB,), jnp.int32),
      pltpu.VMEM((CT,), jnp.int32),
      pltpu.VMEM((CB, D), jnp.float32),
      pltpu.VMEM_SHARED((NP, D), jnp.float32),
      pltpu.SemaphoreType.DMA,
  ]

  @functools.partial(
      pl.kernel, mesh=mesh,
      out_type=jax.ShapeDtypeStruct((NC, NP, D), jnp.float32),
      scratch_types=scratch)
  def k(h_hbm, src_hbm, dst_hbm, z128_hbm, agg_hbm,
        src_v, dst_v, cv, cvt, rows_v, acc, sem):
    c = lax.axis_index("c")
    s = lax.axis_index("s")
    wid = c * NS + s
    pltpu.sync_copy(src_hbm.at[wid], src_v)
    pltpu.sync_copy(dst_hbm.at[wid], dst_v)
    pltpu.sync_copy(z128_hbm, acc.at[pl.ds(s * RPT, RPT)])
    plsc.subcore_barrier()

    @pl.loop(0, NCHB)
    def _(j):
      @pl.loop(0, CB, step=16)
      def _(q):
        cv[pl.ds(q, 16)] = dst_v[pl.ds(j * CB + q, 16)]
      pltpu.async_copy(h_hbm.at[src_v.at[pl.ds(j * CB, CB)]],
                       rows_v, sem).wait()
      pltpu.sync_copy(rows_v, acc.at[cv], add=True)

    @pl.loop(0, CT, step=16)
    def _(q):
      cvt[pl.ds(q, 16)] = dst_v[pl.ds(NCHB * CB + q, 16)]
    rows_t = rows_v.at[pl.ds(0, CT)]
    pltpu.async_copy(h_hbm.at[src_v.at[pl.ds(NCHB * CB, CT)]],
                     rows_t, sem).wait()
    pltpu.sync_copy(rows_t, acc.at[cvt], add=True)
    plsc.subcore_barrier()
    pltpu.sync_copy(acc.at[pl.ds(s * RPT, RPT)],
                    agg_hbm.at[c].at[pl.ds(s * RPT, RPT)])

  return k(h, src3, dst3, z128)


def _tc_linear_stats(agg2, deg2, h, wlt, bl, wrt):
  def body(agg_ref, deg_ref, h_ref, wl_ref, bl_ref, wr_ref,
           z_ref, st_ref, acc_ref):
    i = pl.program_id(0)
    a = agg_ref[0] + agg_ref[1]
    dg = jnp.maximum(deg_ref[0, :, 0:1] + deg_ref[1, :, 0:1], 1.0)
    mean = a / dg
    z = (jnp.dot(mean, wl_ref[...], preferred_element_type=jnp.float32)
         + bl_ref[...]
         + jnp.dot(h_ref[...], wr_ref[...], preferred_element_type=jnp.float32))
    z_ref[...] = z

    @pl.when(i == 0)
    def _():
      acc_ref[...] = jnp.zeros_like(acc_ref)

    acc_ref[0:1, :] += jnp.sum(z, axis=0, keepdims=True)
    acc_ref[1:2, :] += jnp.sum(z * z, axis=0, keepdims=True)
    st_ref[...] = acc_ref[...]

  return pl.pallas_call(
      body,
      grid=(GRID,),
      in_specs=[
          pl.BlockSpec((NC, R, D), lambda i: (0, i, 0)),
          pl.BlockSpec((NC, R, D), lambda i: (0, i, 0)),
          pl.BlockSpec((R, D), lambda i: (i, 0)),
          pl.BlockSpec((D, D), lambda i: (0, 0)),
          pl.BlockSpec((1, D), lambda i: (0, 0)),
          pl.BlockSpec((D, D), lambda i: (0, 0)),
      ],
      out_specs=[
          pl.BlockSpec((R, D), lambda i: (i, 0)),
          pl.BlockSpec((2, D), lambda i: (0, 0)),
      ],
      out_shape=[
          jax.ShapeDtypeStruct((N, D), jnp.float32),
          jax.ShapeDtypeStruct((2, D), jnp.float32),
      ],
      scratch_shapes=[pltpu.VMEM((2, D), jnp.float32)],
  )(agg2, deg2, h, wlt, bl, wrt)


def _tc_bn_relu(z, st, g, be):
  def body(z_ref, st_ref, g_ref, be_ref, o_ref):
    mu = st_ref[0:1, :] * (1.0 / N)
    var = st_ref[1:2, :] * (1.0 / N) - mu * mu
    inv = lax.rsqrt(var + 1e-5)
    o_ref[...] = jnp.maximum(
        g_ref[...] * (z_ref[...] - mu) * inv + be_ref[...], 0.0)

  return pl.pallas_call(
      body,
      grid=(GRID,),
      in_specs=[
          pl.BlockSpec((R, D), lambda i: (i, 0)),
          pl.BlockSpec((2, D), lambda i: (0, 0)),
          pl.BlockSpec((1, D), lambda i: (0, 0)),
          pl.BlockSpec((1, D), lambda i: (0, 0)),
      ],
      out_specs=pl.BlockSpec((R, D), lambda i: (i, 0)),
      out_shape=jax.ShapeDtypeStruct((N, D), jnp.float32),
  )(z, st, g, be)


def _tc_classifier(h, w1t, b1, w2t, b2):
  def body(h_ref, w1_ref, b1_ref, w2_ref, b2_ref, p_ref):
    t = jnp.maximum(
        jnp.dot(h_ref[...], w1_ref[...], preferred_element_type=jnp.float32)
        + b1_ref[...], 0.0)
    lg = (jnp.dot(t, w2_ref[...], preferred_element_type=jnp.float32)
          + b2_ref[...])
    m = jnp.max(lg, axis=1, keepdims=True)
    e = jnp.exp(lg - m)
    p_ref[...] = e / jnp.sum(e, axis=1, keepdims=True)

  return pl.pallas_call(
      body,
      grid=(GRID,),
      in_specs=[
          pl.BlockSpec((R, D), lambda i: (i, 0)),
          pl.BlockSpec((D, 64), lambda i: (0, 0)),
          pl.BlockSpec((1, 64), lambda i: (0, 0)),
          pl.BlockSpec((64, 10), lambda i: (0, 0)),
          pl.BlockSpec((1, 10), lambda i: (0, 0)),
      ],
      out_specs=pl.BlockSpec((R, 10), lambda i: (i, 0)),
      out_shape=jax.ShapeDtypeStruct((N, 10), jnp.float32),
  )(h, w1t, b1, w2t, b2)


def kernel(x, edge_index, Wl0, bl0, Wr0, g0, be0, Wl1, bl1, Wr1, g1, be1,
           Wl2, bl2, Wr2, g2, be2, W1, b1, W2, b2):
  src2 = edge_index[0].reshape(NW, EPW)
  dst2 = edge_index[1].reshape(NW, EPW)
  dst3 = edge_index[1].reshape(NW, NCH, C)
  z128 = jnp.zeros((RPT, D), jnp.float32)
  ones128 = jnp.ones((C, D), jnp.float32)

  deg2 = _sc_degree(dst3, z128, ones128)
  h = x
  for (Wl, bl, Wr, g, be) in ((Wl0, bl0, Wr0, g0, be0),
                              (Wl1, bl1, Wr1, g1, be1),
                              (Wl2, bl2, Wr2, g2, be2)):
    agg2 = _sc_aggregate(h, src2, dst2, z128)
    z, st = _tc_linear_stats(agg2, deg2, h, Wl.T, bl.reshape(1, D),
                             Wr.T)
    h = _tc_bn_relu(z, st, g.reshape(1, D), be.reshape(1, D))

  p = _tc_classifier(h, W1.T, b1.reshape(1, 64), W2.T, b2.reshape(1, 10))
  return (h, p)


# speedup vs baseline: 1.2320x; 1.0289x over previous
import functools

import jax
import jax.numpy as jnp
from jax import lax
from jax.experimental import pallas as pl
from jax.experimental.pallas import tpu as pltpu
from jax.experimental.pallas import tpu_sc as plsc

N = 10000
E = 320000
D = 128
NC = 2
NS = 16
NW = NC * NS
EPW = E // NW
C = 80
NCH = EPW // C
CB = 112
NCHB = EPW // CB
CT = EPW - NCHB * CB
NP = 10240
RPT = NP // NS
R = 1000
GRID = N // R


def _sc_degree(dst3, z128, ones128):
  mesh = plsc.VectorSubcoreMesh(core_axis_name="c", subcore_axis_name="s")
  scratch = [
      pltpu.VMEM((EPW,), jnp.int32),
      pltpu.VMEM((CB,), jnp.int32),
      pltpu.VMEM((CT,), jnp.int32),
      pltpu.VMEM((CB, D), jnp.float32),
      pltpu.VMEM_SHARED((NP, D), jnp.float32),
  ]

  @functools.partial(
      pl.kernel, mesh=mesh,
      out_type=jax.ShapeDtypeStruct((NC, NP, D), jnp.float32),
      scratch_types=scratch)
  def k(dst_hbm, z128_hbm, ones_hbm, deg_hbm, dst_v, cv, cvt, ones_v, dacc):
    c = lax.axis_index("c")
    s = lax.axis_index("s")
    wid = c * NS + s
    pltpu.sync_copy(dst_hbm.at[wid], dst_v)
    pltpu.sync_copy(ones_hbm, ones_v)
    pltpu.sync_copy(z128_hbm, dacc.at[pl.ds(s * RPT, RPT)])
    plsc.subcore_barrier()

    @pl.loop(0, NCHB)
    def _(j):
      @pl.loop(0, CB, step=16)
      def _(q):
        cv[pl.ds(q, 16)] = dst_v[pl.ds(j * CB + q, 16)]
      pltpu.sync_copy(ones_v, dacc.at[cv], add=True)

    @pl.loop(0, CT, step=16)
    def _(q):
      cvt[pl.ds(q, 16)] = dst_v[pl.ds(NCHB * CB + q, 16)]
    pltpu.sync_copy(ones_v.at[pl.ds(0, CT)], dacc.at[cvt], add=True)

    plsc.subcore_barrier()
    pltpu.sync_copy(dacc.at[pl.ds(s * RPT, RPT)],
                    deg_hbm.at[c].at[pl.ds(s * RPT, RPT)])

  return k(dst3, z128, ones128)


def _sc_aggregate(h, src3, dst3, z128):
  mesh = plsc.VectorSubcoreMesh(core_axis_name="c", subcore_axis_name="s")
  scratch = [
      pltpu.VMEM((EPW,), jnp.int32),
      pltpu.VMEM((EPW,), jnp.int32),
      pltpu.VMEM((CB,), jnp.int32),
      pltpu.VMEM((CT,), jnp.int32),
      pltpu.VMEM((CB, D), jnp.float32),
      pltpu.VMEM_SHARED((NP, D), jnp.float32),
      pltpu.SemaphoreType.DMA,
  ]

  @functools.partial(
      pl.kernel, mesh=mesh,
      out_type=jax.ShapeDtypeStruct((NC, NP, D), jnp.float32),
      scratch_types=scratch)
  def k(h_hbm, src_hbm, dst_hbm, z128_hbm, agg_hbm,
        src_v, dst_v, cv, cvt, rows_v, acc, sem):
    c = lax.axis_index("c")
    s = lax.axis_index("s")
    wid = c * NS + s
    pltpu.sync_copy(src_hbm.at[wid], src_v)
    pltpu.sync_copy(dst_hbm.at[wid], dst_v)
    pltpu.sync_copy(z128_hbm, acc.at[pl.ds(s * RPT, RPT)])
    plsc.subcore_barrier()

    @pl.loop(0, NCHB)
    def _(j):
      @pl.loop(0, CB, step=16)
      def _(q):
        cv[pl.ds(q, 16)] = dst_v[pl.ds(j * CB + q, 16)]
      pltpu.async_copy(h_hbm.at[src_v.at[pl.ds(j * CB, CB)]],
                       rows_v, sem).wait()
      pltpu.sync_copy(rows_v, acc.at[cv], add=True)

    @pl.loop(0, CT, step=16)
    def _(q):
      cvt[pl.ds(q, 16)] = dst_v[pl.ds(NCHB * CB + q, 16)]
    rows_t = rows_v.at[pl.ds(0, CT)]
    pltpu.async_copy(h_hbm.at[src_v.at[pl.ds(NCHB * CB, CT)]],
                     rows_t, sem).wait()
    pltpu.sync_copy(rows_t, acc.at[cvt], add=True)
    plsc.subcore_barrier()
    pltpu.sync_copy(acc.at[pl.ds(s * RPT, RPT)],
                    agg_hbm.at[c].at[pl.ds(s * RPT, RPT)])

  return k(h, src3, dst3, z128)


def _tc_layer(agg2, deg2, h, wlt, bl, wrt, g, be, cls=None):
  def body(*refs):
    if cls is None:
      (agg_ref, deg_ref, h_ref, wl_ref, bl_ref, wr_ref, g_ref, be_ref,
       o_ref, zbuf, acc_ref) = refs
    else:
      (agg_ref, deg_ref, h_ref, wl_ref, bl_ref, wr_ref, g_ref, be_ref,
       w1_ref, b1_ref, w2_ref, b2_ref, o_ref, p_ref, zbuf, acc_ref) = refs
    p = pl.program_id(0)
    i = pl.program_id(1)

    @pl.when(p == 0)
    def _():
      a = agg_ref[0] + agg_ref[1]
      dg = jnp.maximum(deg_ref[0, :, 0:1] + deg_ref[1, :, 0:1], 1.0)
      z = (jnp.dot(a / dg, wl_ref[...], preferred_element_type=jnp.float32)
           + bl_ref[...]
           + jnp.dot(h_ref[...], wr_ref[...],
                     preferred_element_type=jnp.float32))
      zbuf[pl.ds(i * R, R), :] = z

      @pl.when(i == 0)
      def _():
        acc_ref[...] = jnp.zeros_like(acc_ref)

      acc_ref[0:1, :] += jnp.sum(z, axis=0, keepdims=True)
      acc_ref[1:2, :] += jnp.sum(z * z, axis=0, keepdims=True)

    @pl.when(p == 1)
    def _():
      mu = acc_ref[0:1, :] * (1.0 / N)
      var = acc_ref[1:2, :] * (1.0 / N) - mu * mu
      inv = lax.rsqrt(var + 1e-5)
      z = zbuf[pl.ds(i * R, R), :]
      hv = jnp.maximum(g_ref[...] * (z - mu) * inv + be_ref[...], 0.0)
      o_ref[...] = hv
      if cls is not None:
        t = jnp.maximum(
            jnp.dot(hv, w1_ref[...], preferred_element_type=jnp.float32)
            + b1_ref[...], 0.0)
        lg = (jnp.dot(t, w2_ref[...], preferred_element_type=jnp.float32)
              + b2_ref[...])
        m = jnp.max(lg, axis=1, keepdims=True)
        e = jnp.exp(lg - m)
        p_ref[...] = e / jnp.sum(e, axis=1, keepdims=True)

  in_specs = [
      pl.BlockSpec((NC, R, D), lambda p, i: (0, i * (1 - p), 0)),
      pl.BlockSpec((NC, R, D), lambda p, i: (0, i * (1 - p), 0)),
      pl.BlockSpec((R, D), lambda p, i: (i * (1 - p), 0)),
      pl.BlockSpec((D, D), lambda p, i: (0, 0)),
      pl.BlockSpec((1, D), lambda p, i: (0, 0)),
      pl.BlockSpec((D, D), lambda p, i: (0, 0)),
      pl.BlockSpec((1, D), lambda p, i: (0, 0)),
      pl.BlockSpec((1, D), lambda p, i: (0, 0)),
  ]
  out_specs = [pl.BlockSpec((R, D), lambda p, i: (i * p, 0))]
  out_shape = [jax.ShapeDtypeStruct((N, D), jnp.float32)]
  args = [agg2, deg2, h, wlt, bl, wrt, g, be]
  if cls is not None:
    w1t, b1, w2t, b2 = cls
    in_specs += [
        pl.BlockSpec((D, 64), lambda p, i: (0, 0)),
        pl.BlockSpec((1, 64), lambda p, i: (0, 0)),
        pl.BlockSpec((64, 10), lambda p, i: (0, 0)),
        pl.BlockSpec((1, 10), lambda p, i: (0, 0)),
    ]
    out_specs += [pl.BlockSpec((R, 10), lambda p, i: (i * p, 0))]
    out_shape += [jax.ShapeDtypeStruct((N, 10), jnp.float32)]
    args += [w1t, b1, w2t, b2]

  out = pl.pallas_call(
      body,
      grid=(2, GRID),
      in_specs=in_specs,
      out_specs=out_specs,
      out_shape=out_shape,
      scratch_shapes=[pltpu.VMEM((N, D), jnp.float32),
                      pltpu.VMEM((2, D), jnp.float32)],
  )(*args)
  return out if cls is not None else out[0]


def kernel(x, edge_index, Wl0, bl0, Wr0, g0, be0, Wl1, bl1, Wr1, g1, be1,
           Wl2, bl2, Wr2, g2, be2, W1, b1, W2, b2):
  src2 = edge_index[0].reshape(NW, EPW)
  dst2 = edge_index[1].reshape(NW, EPW)
  z128 = jnp.zeros((RPT, D), jnp.float32)
  ones128 = jnp.ones((CB, D), jnp.float32)

  deg2 = _sc_degree(dst2, z128, ones128)
  cls = (W1.T, b1.reshape(1, 64), W2.T, b2.reshape(1, 10))
  h = x
  for li, (Wl, bl, Wr, g, be) in enumerate(((Wl0, bl0, Wr0, g0, be0),
                                            (Wl1, bl1, Wr1, g1, be1),
                                            (Wl2, bl2, Wr2, g2, be2))):
    agg2 = _sc_aggregate(h, src2, dst2, z128)
    out = _tc_layer(agg2, deg2, h, Wl.T, bl.reshape(1, D), Wr.T,
                    g.reshape(1, D), be.reshape(1, D),
                    cls=cls if li == 2 else None)
    h = out if li < 2 else out[0]

  return (h, out[1])


# speedup vs baseline: 1.2711x; 1.0318x over previous
import functools

import jax
import jax.numpy as jnp
from jax import lax
from jax.experimental import pallas as pl
from jax.experimental.pallas import tpu as pltpu
from jax.experimental.pallas import tpu_sc as plsc

N = 10000
E = 320000
D = 128
NC = 2
NS = 16
NW = NC * NS
EPW = E // NW
C = 80
NCH = EPW // C
CB = 128
NCHB = EPW // CB
CT = EPW - NCHB * CB
NP = 10240
RPT = NP // NS
R = 1000
GRID = N // R


def _sc_degree(dst3, z128, ones128):
  mesh = plsc.VectorSubcoreMesh(core_axis_name="c", subcore_axis_name="s")
  scratch = [
      pltpu.VMEM((EPW,), jnp.int32),
      pltpu.VMEM((CB,), jnp.int32),
      pltpu.VMEM((CT,), jnp.int32),
      pltpu.VMEM((CB, D), jnp.float32),
      pltpu.VMEM_SHARED((NP, D), jnp.float32),
  ]

  @functools.partial(
      pl.kernel, mesh=mesh,
      out_type=jax.ShapeDtypeStruct((NC, NP, D), jnp.float32),
      scratch_types=scratch)
  def k(dst_hbm, z128_hbm, ones_hbm, deg_hbm, dst_v, cv, cvt, ones_v, dacc):
    c = lax.axis_index("c")
    s = lax.axis_index("s")
    wid = c * NS + s
    pltpu.sync_copy(dst_hbm.at[wid], dst_v)
    pltpu.sync_copy(ones_hbm, ones_v)
    pltpu.sync_copy(z128_hbm, dacc.at[pl.ds(s * RPT, RPT)])
    plsc.subcore_barrier()

    @pl.loop(0, NCHB)
    def _(j):
      @pl.loop(0, CB, step=16)
      def _(q):
        cv[pl.ds(q, 16)] = dst_v[pl.ds(j * CB + q, 16)]
      pltpu.sync_copy(ones_v, dacc.at[cv], add=True)

    @pl.loop(0, CT, step=16)
    def _(q):
      cvt[pl.ds(q, 16)] = dst_v[pl.ds(NCHB * CB + q, 16)]
    pltpu.sync_copy(ones_v.at[pl.ds(0, CT)], dacc.at[cvt], add=True)

    plsc.subcore_barrier()
    pltpu.sync_copy(dacc.at[pl.ds(s * RPT, RPT)],
                    deg_hbm.at[c].at[pl.ds(s * RPT, RPT)])

  return k(dst3, z128, ones128)


def _sc_aggregate(h, src3, dst3, z128):
  mesh = plsc.VectorSubcoreMesh(core_axis_name="c", subcore_axis_name="s")
  scratch = [
      pltpu.VMEM((EPW,), jnp.int32),
      pltpu.VMEM((EPW,), jnp.int32),
      pltpu.VMEM((CB,), jnp.int32),
      pltpu.VMEM((CT,), jnp.int32),
      pltpu.VMEM((CB, D), jnp.float32),
      pltpu.VMEM_SHARED((NP, D), jnp.float32),
      pltpu.SemaphoreType.DMA,
  ]

  @functools.partial(
      pl.kernel, mesh=mesh,
      out_type=jax.ShapeDtypeStruct((NC, NP, D), jnp.float32),
      scratch_types=scratch)
  def k(h_hbm, src_hbm, dst_hbm, z128_hbm, agg_hbm,
        src_v, dst_v, cv, cvt, rows_v, acc, sem):
    c = lax.axis_index("c")
    s = lax.axis_index("s")
    wid = c * NS + s
    pltpu.sync_copy(src_hbm.at[wid], src_v)
    pltpu.sync_copy(dst_hbm.at[wid], dst_v)
    pltpu.sync_copy(z128_hbm, acc.at[pl.ds(s * RPT, RPT)])
    plsc.subcore_barrier()

    @pl.loop(0, NCHB)
    def _(j):
      @pl.loop(0, CB, step=16)
      def _(q):
        cv[pl.ds(q, 16)] = dst_v[pl.ds(j * CB + q, 16)]
      pltpu.async_copy(h_hbm.at[src_v.at[pl.ds(j * CB, CB)]],
                       rows_v, sem).wait()
      pltpu.sync_copy(rows_v, acc.at[cv], add=True)

    @pl.loop(0, CT, step=16)
    def _(q):
      cvt[pl.ds(q, 16)] = dst_v[pl.ds(NCHB * CB + q, 16)]
    rows_t = rows_v.at[pl.ds(0, CT)]
    pltpu.async_copy(h_hbm.at[src_v.at[pl.ds(NCHB * CB, CT)]],
                     rows_t, sem).wait()
    pltpu.sync_copy(rows_t, acc.at[cvt], add=True)
    plsc.subcore_barrier()
    pltpu.sync_copy(acc.at[pl.ds(s * RPT, RPT)],
                    agg_hbm.at[c].at[pl.ds(s * RPT, RPT)])

  return k(h, src3, dst3, z128)


def _tc_layer(agg2, deg2, h, wlt, bl, wrt, g, be, cls=None):
  def body(*refs):
    if cls is None:
      (agg_ref, deg_ref, h_ref, wl_ref, bl_ref, wr_ref, g_ref, be_ref,
       o_ref, zbuf, acc_ref) = refs
    else:
      (agg_ref, deg_ref, h_ref, wl_ref, bl_ref, wr_ref, g_ref, be_ref,
       w1_ref, b1_ref, w2_ref, b2_ref, o_ref, p_ref, zbuf, acc_ref) = refs
    p = pl.program_id(0)
    i = pl.program_id(1)

    @pl.when(p == 0)
    def _():
      a = agg_ref[0] + agg_ref[1]
      dg = jnp.maximum(deg_ref[0, :, 0:1] + deg_ref[1, :, 0:1], 1.0)
      z = (jnp.dot(a / dg, wl_ref[...], preferred_element_type=jnp.float32)
           + bl_ref[...]
           + jnp.dot(h_ref[...], wr_ref[...],
                     preferred_element_type=jnp.float32))
      zbuf[pl.ds(i * R, R), :] = z

      @pl.when(i == 0)
      def _():
        acc_ref[...] = jnp.zeros_like(acc_ref)

      acc_ref[0:1, :] += jnp.sum(z, axis=0, keepdims=True)
      acc_ref[1:2, :] += jnp.sum(z * z, axis=0, keepdims=True)

    @pl.when(p == 1)
    def _():
      mu = acc_ref[0:1, :] * (1.0 / N)
      var = acc_ref[1:2, :] * (1.0 / N) - mu * mu
      inv = lax.rsqrt(var + 1e-5)
      z = zbuf[pl.ds(i * R, R), :]
      hv = jnp.maximum(g_ref[...] * (z - mu) * inv + be_ref[...], 0.0)
      o_ref[...] = hv
      if cls is not None:
        t = jnp.maximum(
            jnp.dot(hv, w1_ref[...], preferred_element_type=jnp.float32)
            + b1_ref[...], 0.0)
        lg = (jnp.dot(t, w2_ref[...], preferred_element_type=jnp.float32)
              + b2_ref[...])
        m = jnp.max(lg, axis=1, keepdims=True)
        e = jnp.exp(lg - m)
        p_ref[...] = e / jnp.sum(e, axis=1, keepdims=True)

  in_specs = [
      pl.BlockSpec((NC, R, D), lambda p, i: (0, i * (1 - p), 0)),
      pl.BlockSpec((NC, R, D), lambda p, i: (0, i * (1 - p), 0)),
      pl.BlockSpec((R, D), lambda p, i: (i * (1 - p), 0)),
      pl.BlockSpec((D, D), lambda p, i: (0, 0)),
      pl.BlockSpec((1, D), lambda p, i: (0, 0)),
      pl.BlockSpec((D, D), lambda p, i: (0, 0)),
      pl.BlockSpec((1, D), lambda p, i: (0, 0)),
      pl.BlockSpec((1, D), lambda p, i: (0, 0)),
  ]
  out_specs = [pl.BlockSpec((R, D), lambda p, i: (i * p, 0))]
  out_shape = [jax.ShapeDtypeStruct((N, D), jnp.float32)]
  args = [agg2, deg2, h, wlt, bl, wrt, g, be]
  if cls is not None:
    w1t, b1, w2t, b2 = cls
    in_specs += [
        pl.BlockSpec((D, 64), lambda p, i: (0, 0)),
        pl.BlockSpec((1, 64), lambda p, i: (0, 0)),
        pl.BlockSpec((64, 10), lambda p, i: (0, 0)),
        pl.BlockSpec((1, 10), lambda p, i: (0, 0)),
    ]
    out_specs += [pl.BlockSpec((R, 10), lambda p, i: (i * p, 0))]
    out_shape += [jax.ShapeDtypeStruct((N, 10), jnp.float32)]
    args += [w1t, b1, w2t, b2]

  out = pl.pallas_call(
      body,
      grid=(2, GRID),
      in_specs=in_specs,
      out_specs=out_specs,
      out_shape=out_shape,
      scratch_shapes=[pltpu.VMEM((N, D), jnp.float32),
                      pltpu.VMEM((2, D), jnp.float32)],
  )(*args)
  return out if cls is not None else out[0]


def kernel(x, edge_index, Wl0, bl0, Wr0, g0, be0, Wl1, bl1, Wr1, g1, be1,
           Wl2, bl2, Wr2, g2, be2, W1, b1, W2, b2):
  src2 = edge_index[0].reshape(NW, EPW)
  dst2 = edge_index[1].reshape(NW, EPW)
  z128 = jnp.zeros((RPT, D), jnp.float32)
  ones128 = jnp.ones((CB, D), jnp.float32)

  deg2 = _sc_degree(dst2, z128, ones128)
  cls = (W1.T, b1.reshape(1, 64), W2.T, b2.reshape(1, 10))
  h = x
  for li, (Wl, bl, Wr, g, be) in enumerate(((Wl0, bl0, Wr0, g0, be0),
                                            (Wl1, bl1, Wr1, g1, be1),
                                            (Wl2, bl2, Wr2, g2, be2))):
    agg2 = _sc_aggregate(h, src2, dst2, z128)
    out = _tc_layer(agg2, deg2, h, Wl.T, bl.reshape(1, D), Wr.T,
                    g.reshape(1, D), be.reshape(1, D),
                    cls=cls if li == 2 else None)
    h = out if li < 2 else out[0]

  return (h, out[1])


# speedup vs baseline: 1.2820x; 1.0086x over previous
import functools

import jax
import jax.numpy as jnp
from jax import lax
from jax.experimental import pallas as pl
from jax.experimental.pallas import tpu as pltpu
from jax.experimental.pallas import tpu_sc as plsc

N = 10000
E = 320000
D = 128
NC = 2
NS = 16
NW = NC * NS
EPW = E // NW
C = 80
NCH = EPW // C
CB = 128
NCHB = EPW // CB
CT = EPW - NCHB * CB
NP = 10240
RPT = NP // NS
R = 1000
GRID = N // R


def _sc_aggregate(h, src2, dst2, z128, ones128=None):
  with_deg = ones128 is not None
  mesh = plsc.VectorSubcoreMesh(core_axis_name="c", subcore_axis_name="s")
  scratch = [
      pltpu.VMEM((EPW,), jnp.int32),
      pltpu.VMEM((EPW,), jnp.int32),
      pltpu.VMEM((CB,), jnp.int32),
      pltpu.VMEM((CT,), jnp.int32),
      pltpu.VMEM((CB, D), jnp.float32),
      pltpu.VMEM_SHARED((NP, D), jnp.float32),
      pltpu.SemaphoreType.DMA,
  ]
  agg_t = jax.ShapeDtypeStruct((NC, NP, D), jnp.float32)
  out_type = [agg_t, agg_t] if with_deg else agg_t

  @functools.partial(pl.kernel, mesh=mesh, out_type=out_type,
                     scratch_types=scratch)
  def k(*refs):
    if with_deg:
      (h_hbm, src_hbm, dst_hbm, z128_hbm, ones_hbm, agg_hbm, deg_hbm,
       src_v, dst_v, cv, cvt, rows_v, acc, sem) = refs
    else:
      (h_hbm, src_hbm, dst_hbm, z128_hbm, agg_hbm,
       src_v, dst_v, cv, cvt, rows_v, acc, sem) = refs
    c = lax.axis_index("c")
    s = lax.axis_index("s")
    wid = c * NS + s
    pltpu.sync_copy(src_hbm.at[wid], src_v)
    pltpu.sync_copy(dst_hbm.at[wid], dst_v)
    pltpu.sync_copy(z128_hbm, acc.at[pl.ds(s * RPT, RPT)])

    def scatter_dst_chunks(src_rows_fn):
      @pl.loop(0, NCHB)
      def _(j):
        @pl.loop(0, CB, step=16)
        def _(q):
          cv[pl.ds(q, 16)] = dst_v[pl.ds(j * CB + q, 16)]
        pltpu.sync_copy(src_rows_fn(j, CB), acc.at[cv], add=True)

      @pl.loop(0, CT, step=16)
      def _(q):
        cvt[pl.ds(q, 16)] = dst_v[pl.ds(NCHB * CB + q, 16)]
      pltpu.sync_copy(src_rows_fn(NCHB, CT), acc.at[cvt], add=True)

    def writeout(dst_hbm_arr):
      pltpu.sync_copy(acc.at[pl.ds(s * RPT, RPT)],
                      dst_hbm_arr.at[c].at[pl.ds(s * RPT, RPT)])

    if with_deg:
      pltpu.sync_copy(ones_hbm, rows_v)
      plsc.subcore_barrier()
      scatter_dst_chunks(lambda j, n: rows_v.at[pl.ds(0, n)]
                         if n != CB else rows_v)
      plsc.subcore_barrier()
      writeout(deg_hbm)
      pltpu.sync_copy(z128_hbm, acc.at[pl.ds(s * RPT, RPT)])
    plsc.subcore_barrier()

    def gather_rows(j, n):
      buf = rows_v.at[pl.ds(0, n)] if n != CB else rows_v
      pltpu.async_copy(h_hbm.at[src_v.at[pl.ds(j * CB, n)]],
                       buf, sem).wait()
      return buf

    scatter_dst_chunks(gather_rows)
    plsc.subcore_barrier()
    writeout(agg_hbm)

  if with_deg:
    return k(h, src2, dst2, z128, ones128)
  return k(h, src2, dst2, z128)


def _tc_layer(agg2, deg2, h, wlt, bl, wrt, g, be, cls=None):
  def body(*refs):
    if cls is None:
      (agg_ref, deg_ref, h_ref, wl_ref, bl_ref, wr_ref, g_ref, be_ref,
       o_ref, zbuf, acc_ref) = refs
    else:
      (agg_ref, deg_ref, h_ref, wl_ref, bl_ref, wr_ref, g_ref, be_ref,
       w1_ref, b1_ref, w2_ref, b2_ref, o_ref, p_ref, zbuf, acc_ref) = refs
    p = pl.program_id(0)
    i = pl.program_id(1)

    @pl.when(p == 0)
    def _():
      a = agg_ref[0] + agg_ref[1]
      dg = jnp.maximum(deg_ref[0, :, 0:1] + deg_ref[1, :, 0:1], 1.0)
      z = (jnp.dot(a / dg, wl_ref[...], preferred_element_type=jnp.float32)
           + bl_ref[...]
           + jnp.dot(h_ref[...], wr_ref[...],
                     preferred_element_type=jnp.float32))
      zbuf[pl.ds(i * R, R), :] = z

      @pl.when(i == 0)
      def _():
        acc_ref[...] = jnp.zeros_like(acc_ref)

      acc_ref[0:1, :] += jnp.sum(z, axis=0, keepdims=True)
      acc_ref[1:2, :] += jnp.sum(z * z, axis=0, keepdims=True)

    @pl.when(p == 1)
    def _():
      mu = acc_ref[0:1, :] * (1.0 / N)
      var = acc_ref[1:2, :] * (1.0 / N) - mu * mu
      inv = lax.rsqrt(var + 1e-5)
      z = zbuf[pl.ds(i * R, R), :]
      hv = jnp.maximum(g_ref[...] * (z - mu) * inv + be_ref[...], 0.0)
      o_ref[...] = hv
      if cls is not None:
        t = jnp.maximum(
            jnp.dot(hv, w1_ref[...], preferred_element_type=jnp.float32)
            + b1_ref[...], 0.0)
        lg = (jnp.dot(t, w2_ref[...], preferred_element_type=jnp.float32)
              + b2_ref[...])
        m = jnp.max(lg, axis=1, keepdims=True)
        e = jnp.exp(lg - m)
        p_ref[...] = e / jnp.sum(e, axis=1, keepdims=True)

  in_specs = [
      pl.BlockSpec((NC, R, D), lambda p, i: (0, i * (1 - p), 0)),
      pl.BlockSpec((NC, R, D), lambda p, i: (0, i * (1 - p), 0)),
      pl.BlockSpec((R, D), lambda p, i: (i * (1 - p), 0)),
      pl.BlockSpec((D, D), lambda p, i: (0, 0)),
      pl.BlockSpec((1, D), lambda p, i: (0, 0)),
      pl.BlockSpec((D, D), lambda p, i: (0, 0)),
      pl.BlockSpec((1, D), lambda p, i: (0, 0)),
      pl.BlockSpec((1, D), lambda p, i: (0, 0)),
  ]
  out_specs = [pl.BlockSpec((R, D), lambda p, i: (i * p, 0))]
  out_shape = [jax.ShapeDtypeStruct((N, D), jnp.float32)]
  args = [agg2, deg2, h, wlt, bl, wrt, g, be]
  if cls is not None:
    w1t, b1, w2t, b2 = cls
    in_specs += [
        pl.BlockSpec((D, 64), lambda p, i: (0, 0)),
        pl.BlockSpec((1, 64), lambda p, i: (0, 0)),
        pl.BlockSpec((64, 10), lambda p, i: (0, 0)),
        pl.BlockSpec((1, 10), lambda p, i: (0, 0)),
    ]
    out_specs += [pl.BlockSpec((R, 10), lambda p, i: (i * p, 0))]
    out_shape += [jax.ShapeDtypeStruct((N, 10), jnp.float32)]
    args += [w1t, b1, w2t, b2]

  out = pl.pallas_call(
      body,
      grid=(2, GRID),
      in_specs=in_specs,
      out_specs=out_specs,
      out_shape=out_shape,
      scratch_shapes=[pltpu.VMEM((N, D), jnp.float32),
                      pltpu.VMEM((2, D), jnp.float32)],
  )(*args)
  return out if cls is not None else out[0]


def kernel(x, edge_index, Wl0, bl0, Wr0, g0, be0, Wl1, bl1, Wr1, g1, be1,
           Wl2, bl2, Wr2, g2, be2, W1, b1, W2, b2):
  src2 = edge_index[0].reshape(NW, EPW)
  dst2 = edge_index[1].reshape(NW, EPW)
  z128 = jnp.zeros((RPT, D), jnp.float32)
  ones128 = jnp.ones((CB, D), jnp.float32)

  cls = (W1.T, b1.reshape(1, 64), W2.T, b2.reshape(1, 10))
  h = x
  for li, (Wl, bl, Wr, g, be) in enumerate(((Wl0, bl0, Wr0, g0, be0),
                                            (Wl1, bl1, Wr1, g1, be1),
                                            (Wl2, bl2, Wr2, g2, be2))):
    if li == 0:
      agg2, deg2 = _sc_aggregate(h, src2, dst2, z128, ones128)
    else:
      agg2 = _sc_aggregate(h, src2, dst2, z128)
    out = _tc_layer(agg2, deg2, h, Wl.T, bl.reshape(1, D), Wr.T,
                    g.reshape(1, D), be.reshape(1, D),
                    cls=cls if li == 2 else None)
    h = out if li < 2 else out[0]

  return (h, out[1])
